# baseline (speedup 1.0000x reference)
import jax
import jax.numpy as jnp
from jax import lax
from jax.experimental import pallas as pl
from jax.experimental.pallas import tpu as pltpu
from jax.experimental.pallas import tpu_sc as plsc

_BATCH = 64
_N_PATCHES = 1024
_MODEL_DIM = 768
_LANES = 16

_NUM_WORKERS = 32
_P_PER_W = _N_PATCHES // _NUM_WORKERS
_VECS_PER_ROW = _MODEL_DIM // _LANES
_NBUF = 2


def _sc_body(patches_hbm, pos_hbm, out_hbm, pos_v, in_bufs, out_bufs,
             pos_sem, in_sems, out_sems):
    nc = 2
    wid = lax.axis_index("s") * nc + lax.axis_index("c")
    p0 = wid * _P_PER_W

    def start_in(b, k):
        pltpu.async_copy(patches_hbm.at[b, pl.ds(p0, _P_PER_W)],
                         in_bufs[k], in_sems[k])

    def wait_in(b, k):
        pltpu.make_async_copy(patches_hbm.at[b, pl.ds(p0, _P_PER_W)],
                              in_bufs[k], in_sems[k]).wait()

    def start_out(b, k):
        pltpu.async_copy(out_bufs[k], out_hbm.at[b, pl.ds(p0, _P_PER_W)],
                         out_sems[k])

    def wait_out(b, k):
        pltpu.make_async_copy(out_bufs[k], out_hbm.at[b, pl.ds(p0, _P_PER_W)],
                              out_sems[k]).wait()

    def compute(k):
        def row_step(r, carry):
            for j in range(_VECS_PER_ROW):
                sl = pl.ds(j * _LANES, _LANES)
                out_bufs[k][r, sl] = in_bufs[k][r, sl] + pos_v[r, sl]
            return carry
        lax.fori_loop(0, _P_PER_W, row_step, 0, unroll=False)

    pltpu.async_copy(pos_hbm.at[pl.ds(p0, _P_PER_W)], pos_v, pos_sem)
    for k in range(_NBUF):
        start_in(k, k)
    pltpu.make_async_copy(pos_hbm.at[pl.ds(p0, _P_PER_W)], pos_v,
                          pos_sem).wait()

    def batch_group(g, carry):
        for k in range(_NBUF):
            b = g + k
            wait_in(b, k)

            @pl.when(g > 0)
            def _():
                wait_out(b - _NBUF, k)

            compute(k)
            start_out(b, k)

            @pl.when(b + _NBUF < _BATCH)
            def _():
                start_in(b + _NBUF, k)
        return carry

    lax.fori_loop(0, _BATCH // _NBUF,
                  lambda i, c: batch_group(i * _NBUF, c), 0, unroll=False)

    for k in range(_NBUF):
        wait_out(_BATCH - _NBUF + k, k)


@jax.jit
def kernel(patches, pos_table):
    mesh = plsc.VectorSubcoreMesh(core_axis_name="c", subcore_axis_name="s")
    return pl.kernel(
        _sc_body,
        out_type=jax.ShapeDtypeStruct((_BATCH, _N_PATCHES, _MODEL_DIM),
                                      jnp.float32),
        mesh=mesh,
        scratch_types=[
            pltpu.VMEM((_P_PER_W, _MODEL_DIM), jnp.float32),
            [pltpu.VMEM((_P_PER_W, _MODEL_DIM), jnp.float32)
             for _ in range(_NBUF)],
            [pltpu.VMEM((_P_PER_W, _MODEL_DIM), jnp.float32)
             for _ in range(_NBUF)],
            pltpu.SemaphoreType.DMA,
            [pltpu.SemaphoreType.DMA for _ in range(_NBUF)],
            [pltpu.SemaphoreType.DMA for _ in range(_NBUF)],
        ],
        name="pos_embed_add_sc",
    )(patches, pos_table)
